# baseline (speedup 1.0000x reference)
import functools

import jax
import jax.numpy as jnp
from jax import lax
from jax.experimental import pallas as pl
from jax.experimental.pallas import tpu as pltpu
from jax.experimental.pallas import tpu_sc as plsc

N_NODES = 10000
N_EDGES = 320000
D = 128
CHUNK = 128
NC = 2
NS = 16
NW = NC * NS
BLK = 8
CPT = 80
NBLK = CPT // BLK
E_PAD = NW * CPT * CHUNK
N_ACC = N_NODES + 16
ROWS_PER_TILE = N_NODES // NS
ZROWS = N_ACC // NS

_MESH = plsc.VectorSubcoreMesh(core_axis_name="c", subcore_axis_name="s")
_OUT_SEG = [jax.ShapeDtypeStruct((NC, NS, ROWS_PER_TILE, D), jnp.float32)]


def _seg_scratch():
    return dict(
        srcb0=pltpu.VMEM((BLK, CHUNK), jnp.int32),
        srcb1=pltpu.VMEM((BLK, CHUNK), jnp.int32),
        dstb0=pltpu.VMEM((BLK, CHUNK), jnp.int32),
        dstb1=pltpu.VMEM((BLK, CHUNK), jnp.int32),
        rows0=pltpu.VMEM((CHUNK, D), jnp.float32),
        rows1=pltpu.VMEM((CHUNK, D), jnp.float32),
        acc_sh=pltpu.VMEM_SHARED((N_ACC, D), jnp.float32),
        sem_i=pltpu.SemaphoreType.DMA,
        sem_g0=pltpu.SemaphoreType.DMA,
        sem_g1=pltpu.SemaphoreType.DMA,
    )


@functools.partial(pl.kernel, mesh=_MESH, out_type=_OUT_SEG,
                   scratch_types=_seg_scratch())
def _sc_segsum(x_hbm, src_hbm, dst_hbm, z128_hbm, out_acc,
               srcb0, srcb1, dstb0, dstb1, rows0, rows1, acc_sh,
               sem_i, sem_g0, sem_g1):
    c = lax.axis_index("c")
    s = lax.axis_index("s")
    wid = s * NC + c
    blk0 = wid * NBLK
    srcb = (srcb0, srcb1)
    dstb = (dstb0, dstb1)
    rows = (rows0, rows1)
    sem_g = (sem_g0, sem_g1)

    pltpu.sync_copy(z128_hbm, acc_sh.at[pl.ds(s * ZROWS, ZROWS)])
    plsc.subcore_barrier()

    def load_idx_sync(b, q):
        pltpu.sync_copy(src_hbm.at[pl.ds((blk0 + b) * BLK, BLK)], srcb[q])
        pltpu.sync_copy(dst_hbm.at[pl.ds((blk0 + b) * BLK, BLK)], dstb[q])

    def start_idx(b, q):
        pltpu.async_copy(src_hbm.at[pl.ds((blk0 + b) * BLK, BLK)],
                         srcb[q], sem_i)
        pltpu.async_copy(dst_hbm.at[pl.ds((blk0 + b) * BLK, BLK)],
                         dstb[q], sem_i)

    def wait_idx(q):
        pltpu.make_async_copy(src_hbm.at[pl.ds(0, BLK)], srcb[q],
                              sem_i).wait()
        pltpu.make_async_copy(dst_hbm.at[pl.ds(0, BLK)], dstb[q],
                              sem_i).wait()

    def start_gather(q, j, p):
        pltpu.async_copy(x_hbm.at[srcb[q].at[j]], rows[p], sem_g[p])

    def wait_gather(q, j, p):
        pltpu.make_async_copy(x_hbm.at[srcb[q].at[j]], rows[p],
                              sem_g[p]).wait()

    load_idx_sync(0, 0)
    start_idx(1, 1)
    start_gather(0, 0, 0)

    def body(bb, carry):
        for qb in range(2):
            b = 2 * bb + qb
            for j in range(BLK):
                p = j & 1
                wait_gather(qb, j, p)
                if j < BLK - 1:
                    start_gather(qb, j + 1, 1 - p)
                else:
                    @pl.when(b < NBLK - 1)
                    def _():
                        wait_idx(1 - qb)
                        start_gather(1 - qb, 0, 1 - p)
                pltpu.sync_copy(rows[p], acc_sh.at[dstb[qb].at[j]],
                                add=True)
            @pl.when(b < NBLK - 2)
            def _():
                start_idx(b + 2, qb)
        return carry

    lax.fori_loop(0, NBLK // 2, body, 0)
    plsc.subcore_barrier()

    row0 = s * ROWS_PER_TILE
    pltpu.sync_copy(acc_sh.at[pl.ds(row0, ROWS_PER_TILE)], out_acc.at[c, s])


@functools.partial(pl.kernel, mesh=_MESH, out_type=_OUT_SEG,
                   scratch_types=dict(
                       dstb0=pltpu.VMEM((BLK, CHUNK), jnp.int32),
                       dstb1=pltpu.VMEM((BLK, CHUNK), jnp.int32),
                       onesb=pltpu.VMEM((CHUNK, D), jnp.float32),
                       acc_sh=pltpu.VMEM_SHARED((N_ACC, D), jnp.float32),
                       sem_i=pltpu.SemaphoreType.DMA,
                   ))
def _sc_counts(dst_hbm, z128_hbm, ones_hbm, out_cnt,
               dstb0, dstb1, onesb, acc_sh, sem_i):
    c = lax.axis_index("c")
    s = lax.axis_index("s")
    wid = s * NC + c
    blk0 = wid * NBLK
    dstb = (dstb0, dstb1)

    pltpu.sync_copy(z128_hbm, acc_sh.at[pl.ds(s * ZROWS, ZROWS)])
    pltpu.sync_copy(ones_hbm, onesb)
    plsc.subcore_barrier()

    def start_idx(b, q):
        pltpu.async_copy(dst_hbm.at[pl.ds((blk0 + b) * BLK, BLK)],
                         dstb[q], sem_i)

    def wait_idx(q):
        pltpu.make_async_copy(dst_hbm.at[pl.ds(0, BLK)], dstb[q],
                              sem_i).wait()

    pltpu.sync_copy(dst_hbm.at[pl.ds(blk0 * BLK, BLK)], dstb[0])
    start_idx(1, 1)

    def body(bb, carry):
        for qb in range(2):
            b = 2 * bb + qb
            for j in range(BLK):
                pltpu.sync_copy(onesb, acc_sh.at[dstb[qb].at[j]], add=True)
            @pl.when(b < NBLK - 1)
            def _():
                wait_idx(1 - qb)
            @pl.when(b < NBLK - 2)
            def _():
                start_idx(b + 2, qb)
        return carry

    lax.fori_loop(0, NBLK // 2, body, 0)
    plsc.subcore_barrier()

    row0 = s * ROWS_PER_TILE
    pltpu.sync_copy(acc_sh.at[pl.ds(row0, ROWS_PER_TILE)], out_cnt.at[c, s])


def _tc_combine_body(relu, acc_ref, cnt_ref, x_ref, wl_ref, wr_ref, b_ref,
                     o_ref):
    a = acc_ref[0] + acc_ref[1]
    cnt = cnt_ref[0, :, 0:1] + cnt_ref[1, :, 0:1]
    agg = a * (1.0 / jnp.maximum(cnt, 1.0))
    y = (jnp.dot(agg, wl_ref[...], preferred_element_type=jnp.float32)
         + jnp.dot(x_ref[...], wr_ref[...], preferred_element_type=jnp.float32)
         + b_ref[...])
    if relu:
        y = jnp.maximum(y, 0.0)
    o_ref[...] = y


def _tc_combine(acc, cnt, x, W_l, W_r, b, relu):
    R = 1000
    return pl.pallas_call(
        functools.partial(_tc_combine_body, relu),
        grid=(N_NODES // R,),
        in_specs=[
            pl.BlockSpec((NC, R, D), lambda r: (0, r, 0)),
            pl.BlockSpec((NC, R, D), lambda r: (0, r, 0)),
            pl.BlockSpec((R, D), lambda r: (r, 0)),
            pl.BlockSpec((D, D), lambda r: (0, 0)),
            pl.BlockSpec((D, D), lambda r: (0, 0)),
            pl.BlockSpec((1, D), lambda r: (0, 0)),
        ],
        out_specs=pl.BlockSpec((R, D), lambda r: (r, 0)),
        out_shape=jax.ShapeDtypeStruct((N_NODES, D), jnp.float32),
    )(acc, cnt, x, W_l, W_r, b)


def kernel(x, edge_index, W_l0, b_l0, W_r0, W_l1, b_l1, W_r1):
    src = edge_index[0].astype(jnp.int32)
    dst = edge_index[1].astype(jnp.int32)
    npad = E_PAD - N_EDGES
    src = jnp.concatenate([src, jnp.zeros((npad,), jnp.int32)])
    junk = N_NODES + (jnp.arange(npad, dtype=jnp.int32) % (N_ACC - N_NODES))
    dst = jnp.concatenate([dst, junk])
    src2 = src.reshape(E_PAD // CHUNK, CHUNK)
    dst2 = dst.reshape(E_PAD // CHUNK, CHUNK)
    z128 = jnp.zeros((ZROWS, D), jnp.float32)
    ones128 = jnp.ones((CHUNK, D), jnp.float32)

    (cnt,) = _sc_counts(dst2, z128, ones128)
    cnt = cnt.reshape(NC, N_NODES, D)
    (acc0,) = _sc_segsum(x, src2, dst2, z128)
    acc0 = acc0.reshape(NC, N_NODES, D)
    h = _tc_combine(acc0, cnt, x, W_l0, W_r0, b_l0.reshape(1, D), relu=True)
    (acc1,) = _sc_segsum(h, src2, dst2, z128)
    acc1 = acc1.reshape(NC, N_NODES, D)
    out = _tc_combine(acc1, cnt, h, W_l1, W_r1, b_l1.reshape(1, D), relu=False)
    return out


# speedup vs baseline: 2.6415x; 2.6415x over previous
import functools

import jax
import jax.numpy as jnp
from jax import lax
from jax.experimental import pallas as pl
from jax.experimental.pallas import tpu as pltpu
from jax.experimental.pallas import tpu_sc as plsc

N_NODES = 10000
N_EDGES = 320000
D = 128
CHUNK = 128
NC = 2
NS = 16
NW = NC * NS
BLK = 8
CPT = 80
NBLK = CPT // BLK
E_PAD = NW * CPT * CHUNK
N_ACC = N_NODES + 16
ROWS_PER_TILE = N_NODES // NS
ZROWS = N_ACC // NS

_MESH = plsc.VectorSubcoreMesh(core_axis_name="c", subcore_axis_name="s")
_OUT_SEG = [jax.ShapeDtypeStruct((NC, NS, ROWS_PER_TILE, D), jnp.float32)]


def _seg_scratch():
    return dict(
        srcb0=pltpu.VMEM((BLK, CHUNK), jnp.int32),
        srcb1=pltpu.VMEM((BLK, CHUNK), jnp.int32),
        dstb0=pltpu.VMEM((BLK, CHUNK), jnp.int32),
        dstb1=pltpu.VMEM((BLK, CHUNK), jnp.int32),
        rows0=pltpu.VMEM((CHUNK, D), jnp.float32),
        rows1=pltpu.VMEM((CHUNK, D), jnp.float32),
        acc_sh=pltpu.VMEM_SHARED((N_ACC, D), jnp.float32),
        sem_i=pltpu.SemaphoreType.DMA,
        sem_g0=pltpu.SemaphoreType.DMA,
        sem_g1=pltpu.SemaphoreType.DMA,
    )


@functools.partial(pl.kernel, mesh=_MESH, out_type=_OUT_SEG,
                   scratch_types=_seg_scratch())
def _sc_segsum(x_hbm, src_hbm, dst_hbm, z128_hbm, out_acc,
               srcb0, srcb1, dstb0, dstb1, rows0, rows1, acc_sh,
               sem_i, sem_g0, sem_g1):
    c = lax.axis_index("c")
    s = lax.axis_index("s")
    wid = s * NC + c
    blk0 = wid * NBLK
    srcb = (srcb0, srcb1)
    dstb = (dstb0, dstb1)
    rows = (rows0, rows1)
    sem_g = (sem_g0, sem_g1)

    pltpu.sync_copy(z128_hbm, acc_sh.at[pl.ds(s * ZROWS, ZROWS)])
    plsc.subcore_barrier()

    def load_idx_sync(b, q):
        pltpu.sync_copy(src_hbm.at[pl.ds((blk0 + b) * BLK, BLK)], srcb[q])
        pltpu.sync_copy(dst_hbm.at[pl.ds((blk0 + b) * BLK, BLK)], dstb[q])

    def start_idx(b, q):
        pltpu.async_copy(src_hbm.at[pl.ds((blk0 + b) * BLK, BLK)],
                         srcb[q], sem_i)
        pltpu.async_copy(dst_hbm.at[pl.ds((blk0 + b) * BLK, BLK)],
                         dstb[q], sem_i)

    def wait_idx(q):
        pltpu.make_async_copy(src_hbm.at[pl.ds(0, BLK)], srcb[q],
                              sem_i).wait()
        pltpu.make_async_copy(dst_hbm.at[pl.ds(0, BLK)], dstb[q],
                              sem_i).wait()

    def start_gather(q, j, p):
        pltpu.async_copy(x_hbm.at[srcb[q].at[j]], rows[p], sem_g[p])

    def wait_gather(q, j, p):
        pltpu.make_async_copy(x_hbm.at[srcb[q].at[j]], rows[p],
                              sem_g[p]).wait()

    load_idx_sync(0, 0)
    start_idx(1, 1)
    start_gather(0, 0, 0)

    def body(bb, carry):
        for qb in range(2):
            b = 2 * bb + qb
            for j in range(BLK):
                p = j & 1
                wait_gather(qb, j, p)
                if j < BLK - 1:
                    start_gather(qb, j + 1, 1 - p)
                else:
                    @pl.when(b < NBLK - 1)
                    def _():
                        wait_idx(1 - qb)
                        start_gather(1 - qb, 0, 1 - p)
                pltpu.sync_copy(rows[p], acc_sh.at[dstb[qb].at[j]],
                                add=True)
            @pl.when(b < NBLK - 2)
            def _():
                start_idx(b + 2, qb)
        return carry

    lax.fori_loop(0, NBLK // 2, body, 0)
    plsc.subcore_barrier()

    row0 = s * ROWS_PER_TILE
    pltpu.sync_copy(acc_sh.at[pl.ds(row0, ROWS_PER_TILE)], out_acc.at[c, s])


@functools.partial(pl.kernel, mesh=_MESH, out_type=_OUT_SEG,
                   scratch_types=dict(
                       dstb0=pltpu.VMEM((BLK, CHUNK), jnp.int32),
                       dstb1=pltpu.VMEM((BLK, CHUNK), jnp.int32),
                       onesb=pltpu.VMEM((CHUNK, D), jnp.float32),
                       acc_sh=pltpu.VMEM_SHARED((N_ACC, D), jnp.float32),
                       sem_i=pltpu.SemaphoreType.DMA,
                   ))
def _sc_counts(dst_hbm, z128_hbm, ones_hbm, out_cnt,
               dstb0, dstb1, onesb, acc_sh, sem_i):
    c = lax.axis_index("c")
    s = lax.axis_index("s")
    wid = s * NC + c
    blk0 = wid * NBLK
    dstb = (dstb0, dstb1)

    pltpu.sync_copy(z128_hbm, acc_sh.at[pl.ds(s * ZROWS, ZROWS)])
    pltpu.sync_copy(ones_hbm, onesb)
    plsc.subcore_barrier()

    def start_idx(b, q):
        pltpu.async_copy(dst_hbm.at[pl.ds((blk0 + b) * BLK, BLK)],
                         dstb[q], sem_i)

    def wait_idx(q):
        pltpu.make_async_copy(dst_hbm.at[pl.ds(0, BLK)], dstb[q],
                              sem_i).wait()

    pltpu.sync_copy(dst_hbm.at[pl.ds(blk0 * BLK, BLK)], dstb[0])
    start_idx(1, 1)

    def body(bb, carry):
        for qb in range(2):
            b = 2 * bb + qb
            for j in range(BLK):
                pltpu.sync_copy(onesb, acc_sh.at[dstb[qb].at[j]], add=True)
            @pl.when(b < NBLK - 1)
            def _():
                wait_idx(1 - qb)
            @pl.when(b < NBLK - 2)
            def _():
                start_idx(b + 2, qb)
        return carry

    lax.fori_loop(0, NBLK // 2, body, 0)
    plsc.subcore_barrier()

    row0 = s * ROWS_PER_TILE
    pltpu.sync_copy(acc_sh.at[pl.ds(row0, ROWS_PER_TILE)], out_cnt.at[c, s])


def _tc_combine_body(relu, acc_ref, cnt_ref, x_ref, wl_ref, wr_ref, b_ref,
                     o_ref):
    a = acc_ref[0] + acc_ref[1]
    cnt = cnt_ref[0, :, 0:1] + cnt_ref[1, :, 0:1]
    agg = a * (1.0 / jnp.maximum(cnt, 1.0))
    y = (jnp.dot(agg, wl_ref[...], preferred_element_type=jnp.float32)
         + jnp.dot(x_ref[...], wr_ref[...], preferred_element_type=jnp.float32)
         + b_ref[...])
    if relu:
        y = jnp.maximum(y, 0.0)
    o_ref[...] = y


def _tc_combine(acc, cnt, x, W_l, W_r, b, relu):
    R = 1000
    return pl.pallas_call(
        functools.partial(_tc_combine_body, relu),
        grid=(N_NODES // R,),
        in_specs=[
            pl.BlockSpec((NC, R, D), lambda r: (0, r, 0)),
            pl.BlockSpec((NC, R, D), lambda r: (0, r, 0)),
            pl.BlockSpec((R, D), lambda r: (r, 0)),
            pl.BlockSpec((D, D), lambda r: (0, 0)),
            pl.BlockSpec((D, D), lambda r: (0, 0)),
            pl.BlockSpec((1, D), lambda r: (0, 0)),
        ],
        out_specs=pl.BlockSpec((R, D), lambda r: (r, 0)),
        out_shape=jax.ShapeDtypeStruct((N_NODES, D), jnp.float32),
    )(acc, cnt, x, W_l, W_r, b)


def kernel(x, edge_index, W_l0, b_l0, W_r0, W_l1, b_l1, W_r1):
    src = edge_index[0].astype(jnp.int32)
    dst = edge_index[1].astype(jnp.int32)
    npad = E_PAD - N_EDGES
    jsrc = (jnp.arange(npad, dtype=jnp.int32) * 37) % N_NODES
    src = jnp.concatenate([src, jsrc])
    junk = N_NODES + (jnp.arange(npad, dtype=jnp.int32) % (N_ACC - N_NODES))
    dst = jnp.concatenate([dst, junk])
    src2 = src.reshape(E_PAD // CHUNK, CHUNK)
    dst2 = dst.reshape(E_PAD // CHUNK, CHUNK)
    z128 = jnp.zeros((ZROWS, D), jnp.float32)
    ones128 = jnp.ones((CHUNK, D), jnp.float32)

    (cnt,) = _sc_counts(dst2, z128, ones128)
    cnt = cnt.reshape(NC, N_NODES, D)
    (acc0,) = _sc_segsum(x, src2, dst2, z128)
    acc0 = acc0.reshape(NC, N_NODES, D)
    h = _tc_combine(acc0, cnt, x, W_l0, W_r0, b_l0.reshape(1, D), relu=True)
    (acc1,) = _sc_segsum(h, src2, dst2, z128)
    acc1 = acc1.reshape(NC, N_NODES, D)
    out = _tc_combine(acc1, cnt, h, W_l1, W_r1, b_l1.reshape(1, D), relu=False)
    return out
